# Initial kernel scaffold; baseline (speedup 1.0000x reference)
import jax
import jax.numpy as jnp
from jax.experimental import pallas as pl


def kernel(x, edge_index, edge_attr, conditions, scale, batch, ne_W1, ne_b1, ne_W2, ne_b2, ce_W1, ce_b1, ce_W2, ce_b2, k_W1, k_b1, k_W2, k_b2, k_W3, k_b3, root, conv_b, fo_W, fo_b):
    raise NotImplementedError("write your pallas kernel here")


# speedup vs baseline: 4.5121x; 4.5121x over previous
import functools

import jax
import jax.numpy as jnp
from jax import lax
from jax.experimental import pallas as pl
from jax.experimental.pallas import tpu as pltpu
from jax.experimental.pallas import tpu_sc as plsc

_NC = 2
_NS = 16
_NW = _NC * _NS


def _gather_rows(table, idx2d, n_rows, group, sub):
    T, D = table.shape
    G = group
    mesh = plsc.VectorSubcoreMesh(core_axis_name="c", subcore_axis_name="s")

    def body(table_hbm, idx_hbm, out_hbm, idx_st, rows_v, sem):
        c = lax.axis_index("c")
        s = lax.axis_index("s")
        w = s * _NC + c

        def grp(g, carry):
            chunk0 = (w * G + g) * 8
            pltpu.sync_copy(idx_hbm.at[pl.ds(chunk0, 8)], idx_st)
            descs = [
                pltpu.async_copy(
                    table_hbm.at[idx_st.at[j]],
                    rows_v.at[pl.ds(j * sub, sub)],
                    sem,
                )
                for j in range(8)
            ]
            for d in descs:
                d.wait()
            pltpu.sync_copy(rows_v, out_hbm.at[pl.ds(chunk0 * sub, 8 * sub)])
            return carry

        lax.fori_loop(0, G, grp, 0)

    return pl.kernel(
        body,
        out_type=jax.ShapeDtypeStruct((n_rows, D), jnp.float32),
        mesh=mesh,
        scratch_types=[
            pltpu.VMEM((8, sub), jnp.int32),
            pltpu.VMEM((8 * sub, D), jnp.float32),
            pltpu.SemaphoreType.DMA,
        ],
        compiler_params=pltpu.CompilerParams(use_tc_tiling_on_sc=False),
    )(table, idx2d)


def _scatter_partials(vals, idx2d, n_seg, group, sub, zch, nz):
    E_, D = vals.shape
    G = group
    rpt = n_seg // _NS
    mesh = plsc.VectorSubcoreMesh(core_axis_name="c", subcore_axis_name="s")

    def body(vals_hbm, idx_hbm, out_hbm, idx_st, vals_st, zbuf, acc_sh, sem):
        c = lax.axis_index("c")
        s = lax.axis_index("s")

        def zf(r, carry):
            zbuf[r, :] = jnp.zeros((D,), jnp.float32)
            return carry

        lax.fori_loop(0, zch, zf, 0)
        for k in range(nz):
            pltpu.sync_copy(zbuf, acc_sh.at[pl.ds(s * rpt + k * zch, zch)])
        plsc.subcore_barrier()

        def grp(g, carry):
            chunk0 = ((c * _NS + s) * G + g) * 8
            pltpu.sync_copy(idx_hbm.at[pl.ds(chunk0, 8)], idx_st)
            pltpu.sync_copy(vals_hbm.at[pl.ds(chunk0 * sub, 8 * sub)], vals_st)
            for j in range(8):
                pltpu.sync_copy(
                    vals_st.at[pl.ds(j * sub, sub)],
                    acc_sh.at[idx_st.at[j]],
                    add=True,
                )
            return carry

        lax.fori_loop(0, G, grp, 0)
        plsc.subcore_barrier()
        for k in range(nz):
            pltpu.sync_copy(acc_sh.at[pl.ds(s * rpt + k * zch, zch)], zbuf)
            pltpu.sync_copy(zbuf, out_hbm.at[c, pl.ds(s * rpt + k * zch, zch)])

    return pl.kernel(
        body,
        out_type=jax.ShapeDtypeStruct((_NC, n_seg, D), jnp.float32),
        mesh=mesh,
        scratch_types=[
            pltpu.VMEM((8, sub), jnp.int32),
            pltpu.VMEM((8 * sub, D), jnp.float32),
            pltpu.VMEM((zch, D), jnp.float32),
            pltpu.VMEM_SHARED((n_seg, D), jnp.float32),
            pltpu.SemaphoreType.DMA,
        ],
        compiler_params=pltpu.CompilerParams(use_tc_tiling_on_sc=False),
    )(vals, idx2d)


def _degree_partials(idx2d, n_seg, group, sub, zch, nz, d):
    G = group
    rpt = n_seg // _NS
    mesh = plsc.VectorSubcoreMesh(core_axis_name="c", subcore_axis_name="s")

    def body(idx_hbm, out_hbm, idx_st, ones_v, zbuf, acc_sh, sem):
        c = lax.axis_index("c")
        s = lax.axis_index("s")

        def of(r, carry):
            ones_v[r, :] = jnp.ones((d,), jnp.float32)
            return carry

        lax.fori_loop(0, sub, of, 0)

        def zf(r, carry):
            zbuf[r, :] = jnp.zeros((d,), jnp.float32)
            return carry

        lax.fori_loop(0, zch, zf, 0)
        for k in range(nz):
            pltpu.sync_copy(zbuf, acc_sh.at[pl.ds(s * rpt + k * zch, zch)])
        plsc.subcore_barrier()

        def grp(g, carry):
            chunk0 = ((c * _NS + s) * G + g) * 8
            pltpu.sync_copy(idx_hbm.at[pl.ds(chunk0, 8)], idx_st)
            for j in range(8):
                pltpu.sync_copy(ones_v, acc_sh.at[idx_st.at[j]], add=True)
            return carry

        lax.fori_loop(0, G, grp, 0)
        plsc.subcore_barrier()
        for k in range(nz):
            pltpu.sync_copy(acc_sh.at[pl.ds(s * rpt + k * zch, zch)], zbuf)
            pltpu.sync_copy(zbuf, out_hbm.at[c, pl.ds(s * rpt + k * zch, zch)])

    return pl.kernel(
        body,
        out_type=jax.ShapeDtypeStruct((_NC, n_seg, d), jnp.float32),
        mesh=mesh,
        scratch_types=[
            pltpu.VMEM((8, sub), jnp.int32),
            pltpu.VMEM((sub, d), jnp.float32),
            pltpu.VMEM((zch, d), jnp.float32),
            pltpu.VMEM_SHARED((n_seg, d), jnp.float32),
            pltpu.SemaphoreType.DMA,
        ],
        compiler_params=pltpu.CompilerParams(use_tc_tiling_on_sc=False),
    )(idx2d)


def _tc_cond(cs, W1, b1, W2, b2):
    def body(cs_ref, W1_ref, b1_ref, W2_ref, b2_ref, u_ref):
        t = jnp.maximum(cs_ref[...] @ W1_ref[...] + b1_ref[...], 0.0)
        u_ref[...] = t @ W2_ref[...] + b2_ref[...]

    B, _ = cs.shape
    H = W1.shape[1]
    return pl.pallas_call(
        body, out_shape=jax.ShapeDtypeStruct((B, H), jnp.float32)
    )(cs, W1, b1, W2, b2)


def _tc_nodes(x, W1, b1, W2, b2, nb):
    N_, F = x.shape
    H = W1.shape[1]

    def body(x_ref, W1_ref, b1_ref, W2_ref, b2_ref, h_ref):
        t = jnp.maximum(x_ref[...] @ W1_ref[...] + b1_ref[...], 0.0)
        h_ref[...] = t @ W2_ref[...] + b2_ref[...]

    full = lambda i: (0, 0)
    return pl.pallas_call(
        body,
        grid=(N_ // nb,),
        in_specs=[
            pl.BlockSpec((nb, F), lambda i: (i, 0)),
            pl.BlockSpec(W1.shape, full),
            pl.BlockSpec(b1.shape, full),
            pl.BlockSpec(W2.shape, full),
            pl.BlockSpec(b2.shape, full),
        ],
        out_specs=pl.BlockSpec((nb, H), lambda i: (i, 0)),
        out_shape=jax.ShapeDtypeStruct((N_, H), jnp.float32),
    )(x, W1, b1, W2, b2)


def _tc_msg(ea, ue, xj, W1a, W1b, b1, W2, b2, W3, b3, M, Q, eb):
    E_, A = ea.shape
    H = xj.shape[1]

    def body(ea_ref, ue_ref, xj_ref, W1a_ref, W1b_ref, b1_ref, W2_ref,
             b2_ref, W3_ref, b3_ref, M_ref, Q_ref, msg_ref):
        kh = jnp.maximum(
            ea_ref[...] @ W1a_ref[...] + ue_ref[...] @ W1b_ref[...]
            + b1_ref[...], 0.0)
        kh = jnp.maximum(kh @ W2_ref[...] + b2_ref[...], 0.0)
        kw = kh @ W3_ref[...] + b3_ref[...]
        xe = xj_ref[...] @ M_ref[...]
        msg_ref[...] = (xe * kw) @ Q_ref[...]

    full = lambda i: (0, 0)
    return pl.pallas_call(
        body,
        grid=(E_ // eb,),
        in_specs=[
            pl.BlockSpec((eb, A), lambda i: (i, 0)),
            pl.BlockSpec((eb, H), lambda i: (i, 0)),
            pl.BlockSpec((eb, H), lambda i: (i, 0)),
            pl.BlockSpec(W1a.shape, full),
            pl.BlockSpec(W1b.shape, full),
            pl.BlockSpec(b1.shape, full),
            pl.BlockSpec(W2.shape, full),
            pl.BlockSpec(b2.shape, full),
            pl.BlockSpec(W3.shape, full),
            pl.BlockSpec(b3.shape, full),
            pl.BlockSpec(M.shape, full),
            pl.BlockSpec(Q.shape, full),
        ],
        out_specs=pl.BlockSpec((eb, H), lambda i: (i, 0)),
        out_shape=jax.ShapeDtypeStruct((E_, H), jnp.float32),
    )(ea, ue, xj, W1a, W1b, b1, W2, b2, W3, b3, M, Q)


def _tc_update(h, aggp, degp, root, cb, nb):
    N_, H = h.shape

    def body(h_ref, a_ref, d_ref, root_ref, cb_ref, o_ref):
        agg = a_ref[0] + a_ref[1]
        deg = jnp.maximum(d_ref[0] + d_ref[1], 1.0)
        o_ref[...] = jnp.maximum(
            agg / deg + h_ref[...] @ root_ref[...] + cb_ref[...], 0.0)

    full = lambda i: (0, 0)
    return pl.pallas_call(
        body,
        grid=(N_ // nb,),
        in_specs=[
            pl.BlockSpec((nb, H), lambda i: (i, 0)),
            pl.BlockSpec((2, nb, H), lambda i: (0, i, 0)),
            pl.BlockSpec((2, nb, H), lambda i: (0, i, 0)),
            pl.BlockSpec(root.shape, full),
            pl.BlockSpec(cb.shape, full),
        ],
        out_specs=pl.BlockSpec((nb, H), lambda i: (i, 0)),
        out_shape=jax.ShapeDtypeStruct((N_, H), jnp.float32),
    )(h, aggp, degp, root, cb)


def _tc_update_final(h, aggp, degp, root, cb, foW, fob, nb):
    N_, H = h.shape
    O = foW.shape[1]

    def body(h_ref, a_ref, d_ref, root_ref, cb_ref, foW_ref, fob_ref, o_ref):
        agg = a_ref[0] + a_ref[1]
        deg = jnp.maximum(d_ref[0] + d_ref[1], 1.0)
        hn = jnp.maximum(
            agg / deg + h_ref[...] @ root_ref[...] + cb_ref[...], 0.0)
        o_ref[...] = hn @ foW_ref[...] + fob_ref[...]

    full = lambda i: (0, 0)
    return pl.pallas_call(
        body,
        grid=(N_ // nb,),
        in_specs=[
            pl.BlockSpec((nb, H), lambda i: (i, 0)),
            pl.BlockSpec((2, nb, H), lambda i: (0, i, 0)),
            pl.BlockSpec((2, nb, H), lambda i: (0, i, 0)),
            pl.BlockSpec(root.shape, full),
            pl.BlockSpec(cb.shape, full),
            pl.BlockSpec(foW.shape, full),
            pl.BlockSpec(fob.shape, full),
        ],
        out_specs=pl.BlockSpec((nb, O), lambda i: (i, 0)),
        out_shape=jax.ShapeDtypeStruct((N_, O), jnp.float32),
    )(h, aggp, degp, root, cb, foW, fob)


def kernel(x, edge_index, edge_attr, conditions, scale, batch,
           ne_W1, ne_b1, ne_W2, ne_b2,
           ce_W1, ce_b1, ce_W2, ce_b2,
           k_W1, k_b1, k_W2, k_b2, k_W3, k_b3,
           root, conv_b, fo_W, fo_b):
    N, _ = x.shape
    E = edge_index.shape[1]
    A = edge_attr.shape[1]
    H = ne_W1.shape[1]
    num_layers = 3

    e_sub, e_group = 125, E // (_NW * 8 * 125)
    npad = 51200
    n_sub, n_group = 100, npad // (_NW * 8 * 100)
    zch, nz = 625, (N // _NS) // 625
    nb = 2000
    eb = 4000

    f32 = jnp.float32
    src = edge_index[0]
    dst = edge_index[1]
    src2d = src.reshape(-1, e_sub)
    dst2d = dst.reshape(-1, e_sub)
    batch_p = jnp.pad(batch, (0, npad - N)).reshape(-1, n_sub)

    r1 = lambda b: b.reshape(1, -1)
    cs = jnp.concatenate([conditions, scale], axis=1)

    M = jnp.kron(jnp.eye(H, dtype=f32), jnp.ones((1, H), f32))
    Q = jnp.kron(jnp.ones((H, 1), f32), jnp.eye(H, dtype=f32))

    u = _tc_cond(cs, ce_W1, r1(ce_b1), ce_W2, r1(ce_b2))
    h = _tc_nodes(x, ne_W1, r1(ne_b1), ne_W2, r1(ne_b2), nb)

    ubig = _gather_rows(u, batch_p, npad, n_group, n_sub)
    ue = _gather_rows(ubig, src2d, E, e_group, e_sub)
    degp = _degree_partials(dst2d, N, e_group, e_sub, zch, nz, H)

    W1a, W1b = k_W1[:A], k_W1[A:]
    for layer in range(num_layers):
        xj = _gather_rows(h, src2d, E, e_group, e_sub)
        msg = _tc_msg(edge_attr, ue, xj, W1a, W1b, r1(k_b1), k_W2, r1(k_b2),
                      k_W3, r1(k_b3), M, Q, eb)
        aggp = _scatter_partials(msg, dst2d, N, e_group, e_sub, zch, nz)
        if layer < num_layers - 1:
            h = _tc_update(h, aggp, degp, root, r1(conv_b), nb)
        else:
            out = _tc_update_final(h, aggp, degp, root, r1(conv_b),
                                   fo_W, r1(fo_b), nb)
    return (out, u)


# speedup vs baseline: 5.9142x; 1.3108x over previous
import functools

import jax
import jax.numpy as jnp
from jax import lax
from jax.experimental import pallas as pl
from jax.experimental.pallas import tpu as pltpu
from jax.experimental.pallas import tpu_sc as plsc

_NC = 2
_NS = 16
_NW = _NC * _NS


def _gather_rows(table, idx2d, n_rows, group, sub):
    T, D = table.shape
    G = group
    mesh = plsc.VectorSubcoreMesh(core_axis_name="c", subcore_axis_name="s")

    def body(table_hbm, idx_hbm, out_hbm, idx_st, rows_v, sem):
        c = lax.axis_index("c")
        s = lax.axis_index("s")
        w = s * _NC + c

        def grp(g, carry):
            chunk0 = (w * G + g) * 8
            pltpu.sync_copy(idx_hbm.at[pl.ds(chunk0, 8)], idx_st)
            descs = [
                pltpu.async_copy(
                    table_hbm.at[idx_st.at[j]],
                    rows_v.at[pl.ds(j * sub, sub)],
                    sem,
                )
                for j in range(8)
            ]
            for d in descs:
                d.wait()
            pltpu.sync_copy(rows_v, out_hbm.at[pl.ds(chunk0 * sub, 8 * sub)])
            return carry

        lax.fori_loop(0, G, grp, 0)

    return pl.kernel(
        body,
        out_type=jax.ShapeDtypeStruct((n_rows, D), jnp.float32),
        mesh=mesh,
        scratch_types=[
            pltpu.VMEM((8, sub), jnp.int32),
            pltpu.VMEM((8 * sub, D), jnp.float32),
            pltpu.SemaphoreType.DMA,
        ],
        compiler_params=pltpu.CompilerParams(use_tc_tiling_on_sc=False),
    )(table, idx2d)


def _scatter_partials(vals, idx2d, n_seg, group, sub, zch, nz):
    E_, D = vals.shape
    G = group
    rpt = n_seg // _NS
    mesh = plsc.VectorSubcoreMesh(core_axis_name="c", subcore_axis_name="s")

    def body(vals_hbm, idx_hbm, out_hbm, idx_st, vals_st, zbuf, acc_sh, sem):
        c = lax.axis_index("c")
        s = lax.axis_index("s")

        def zf(r, carry):
            zbuf[r, :] = jnp.zeros((D,), jnp.float32)
            return carry

        lax.fori_loop(0, zch, zf, 0)
        for k in range(nz):
            pltpu.sync_copy(zbuf, acc_sh.at[pl.ds(s * rpt + k * zch, zch)])
        plsc.subcore_barrier()

        def grp(g, carry):
            chunk0 = ((c * _NS + s) * G + g) * 8
            pltpu.sync_copy(idx_hbm.at[pl.ds(chunk0, 8)], idx_st)
            pltpu.sync_copy(vals_hbm.at[pl.ds(chunk0 * sub, 8 * sub)], vals_st)
            for j in range(8):
                pltpu.sync_copy(
                    vals_st.at[pl.ds(j * sub, sub)],
                    acc_sh.at[idx_st.at[j]],
                    add=True,
                )
            return carry

        lax.fori_loop(0, G, grp, 0)
        plsc.subcore_barrier()
        for k in range(nz):
            pltpu.sync_copy(acc_sh.at[pl.ds(s * rpt + k * zch, zch)], zbuf)
            pltpu.sync_copy(zbuf, out_hbm.at[c, pl.ds(s * rpt + k * zch, zch)])

    return pl.kernel(
        body,
        out_type=jax.ShapeDtypeStruct((_NC, n_seg, D), jnp.float32),
        mesh=mesh,
        scratch_types=[
            pltpu.VMEM((8, sub), jnp.int32),
            pltpu.VMEM((8 * sub, D), jnp.float32),
            pltpu.VMEM((zch, D), jnp.float32),
            pltpu.VMEM_SHARED((n_seg, D), jnp.float32),
            pltpu.SemaphoreType.DMA,
        ],
        compiler_params=pltpu.CompilerParams(use_tc_tiling_on_sc=False),
    )(vals, idx2d)


def _degree_partials(idx2d, n_seg, group, sub, zch, nz, d):
    G = group
    rpt = n_seg // _NS
    mesh = plsc.VectorSubcoreMesh(core_axis_name="c", subcore_axis_name="s")

    def body(idx_hbm, out_hbm, idx_st, ones_v, zbuf, acc_sh, sem):
        c = lax.axis_index("c")
        s = lax.axis_index("s")

        def of(r, carry):
            ones_v[r, :] = jnp.ones((d,), jnp.float32)
            return carry

        lax.fori_loop(0, sub, of, 0)

        def zf(r, carry):
            zbuf[r, :] = jnp.zeros((d,), jnp.float32)
            return carry

        lax.fori_loop(0, zch, zf, 0)
        for k in range(nz):
            pltpu.sync_copy(zbuf, acc_sh.at[pl.ds(s * rpt + k * zch, zch)])
        plsc.subcore_barrier()

        def grp(g, carry):
            chunk0 = ((c * _NS + s) * G + g) * 8
            pltpu.sync_copy(idx_hbm.at[pl.ds(chunk0, 8)], idx_st)
            for j in range(8):
                pltpu.sync_copy(ones_v, acc_sh.at[idx_st.at[j]], add=True)
            return carry

        lax.fori_loop(0, G, grp, 0)
        plsc.subcore_barrier()
        for k in range(nz):
            pltpu.sync_copy(acc_sh.at[pl.ds(s * rpt + k * zch, zch)], zbuf)
            pltpu.sync_copy(zbuf, out_hbm.at[c, pl.ds(s * rpt + k * zch, zch)])

    return pl.kernel(
        body,
        out_type=jax.ShapeDtypeStruct((_NC, n_seg, d), jnp.float32),
        mesh=mesh,
        scratch_types=[
            pltpu.VMEM((8, sub), jnp.int32),
            pltpu.VMEM((sub, d), jnp.float32),
            pltpu.VMEM((zch, d), jnp.float32),
            pltpu.VMEM_SHARED((n_seg, d), jnp.float32),
            pltpu.SemaphoreType.DMA,
        ],
        compiler_params=pltpu.CompilerParams(use_tc_tiling_on_sc=False),
    )(idx2d)


def _tc_cond(cs, W1, b1, W2, b2):
    def body(cs_ref, W1_ref, b1_ref, W2_ref, b2_ref, u_ref):
        t = jnp.maximum(cs_ref[...] @ W1_ref[...] + b1_ref[...], 0.0)
        u_ref[...] = t @ W2_ref[...] + b2_ref[...]

    B, _ = cs.shape
    H = W1.shape[1]
    return pl.pallas_call(
        body, out_shape=jax.ShapeDtypeStruct((B, H), jnp.float32)
    )(cs, W1, b1, W2, b2)


def _tc_nodes(x8, W1, b1, W2, b2, rb):
    R, F8 = x8.shape
    F = F8 // 8
    H = W1.shape[1]

    def body(x_ref, W1_ref, b1_ref, W2_ref, b2_ref, h_ref):
        xb = x_ref[...]
        for m in range(8):
            xm = xb[:, F * m:F * (m + 1)]
            t = jnp.maximum(xm @ W1_ref[...] + b1_ref[...], 0.0)
            h_ref[:, H * m:H * (m + 1)] = t @ W2_ref[...] + b2_ref[...]

    full = lambda i: (0, 0)
    return pl.pallas_call(
        body,
        grid=(R // rb,),
        in_specs=[
            pl.BlockSpec((rb, F8), lambda i: (i, 0)),
            pl.BlockSpec(W1.shape, full),
            pl.BlockSpec(b1.shape, full),
            pl.BlockSpec(W2.shape, full),
            pl.BlockSpec(b2.shape, full),
        ],
        out_specs=pl.BlockSpec((rb, 8 * H), lambda i: (i, 0)),
        out_shape=jax.ShapeDtypeStruct((R, 8 * H), jnp.float32),
    )(x8, W1, b1, W2, b2)


def _tc_msg(eag, ue8, xj8, W1a, W1b, b1, W2, b2, W3, b3, M, Q, ebr):
    R, A8 = eag.shape
    A = A8 // 8
    H = W1b.shape[0]

    def body(ea_ref, ue_ref, xj_ref, W1a_ref, W1b_ref, b1_ref, W2_ref,
             b2_ref, W3_ref, b3_ref, M_ref, Q_ref, msg_ref):
        eab = ea_ref[...]
        ueb = ue_ref[...]
        xjb = xj_ref[...]
        for m in range(8):
            eam = eab[:, A * m:A * (m + 1)]
            uem = ueb[:, H * m:H * (m + 1)]
            xjm = xjb[:, H * m:H * (m + 1)]
            kh = jnp.maximum(
                eam @ W1a_ref[...] + uem @ W1b_ref[...] + b1_ref[...], 0.0)
            kh = jnp.maximum(kh @ W2_ref[...] + b2_ref[...], 0.0)
            kw = kh @ W3_ref[...] + b3_ref[...]
            xe = xjm @ M_ref[...]
            msg_ref[:, H * m:H * (m + 1)] = (xe * kw) @ Q_ref[...]

    full = lambda i: (0, 0)
    return pl.pallas_call(
        body,
        grid=(R // ebr,),
        in_specs=[
            pl.BlockSpec((ebr, A8), lambda i: (i, 0)),
            pl.BlockSpec((ebr, 8 * H), lambda i: (i, 0)),
            pl.BlockSpec((ebr, 8 * H), lambda i: (i, 0)),
            pl.BlockSpec(W1a.shape, full),
            pl.BlockSpec(W1b.shape, full),
            pl.BlockSpec(b1.shape, full),
            pl.BlockSpec(W2.shape, full),
            pl.BlockSpec(b2.shape, full),
            pl.BlockSpec(W3.shape, full),
            pl.BlockSpec(b3.shape, full),
            pl.BlockSpec(M.shape, full),
            pl.BlockSpec(Q.shape, full),
        ],
        out_specs=pl.BlockSpec((ebr, 8 * H), lambda i: (i, 0)),
        out_shape=jax.ShapeDtypeStruct((R, 8 * H), jnp.float32),
    )(eag, ue8, xj8, W1a, W1b, b1, W2, b2, W3, b3, M, Q)


def _tc_update(h8, aggp8, degp8, root, cb):
    R, H8 = h8.shape
    H = H8 // 8

    def body(h_ref, a_ref, d_ref, root_ref, cb_ref, o_ref):
        agg = a_ref[0] + a_ref[1]
        deg = jnp.maximum(d_ref[0] + d_ref[1], 1.0)
        rat = agg / deg
        hb = h_ref[...]
        for m in range(8):
            hm = hb[:, H * m:H * (m + 1)]
            am = rat[:, H * m:H * (m + 1)]
            o_ref[:, H * m:H * (m + 1)] = jnp.maximum(
                hm @ root_ref[...] + cb_ref[...] + am, 0.0)

    return pl.pallas_call(
        body,
        out_shape=jax.ShapeDtypeStruct((R, H8), jnp.float32),
    )(h8, aggp8, degp8, root, cb)


def _tc_update_final(h8, aggp8, degp8, root, cb, foW, fob):
    R, H8 = h8.shape
    H = H8 // 8
    O = foW.shape[1]

    def body(h_ref, a_ref, d_ref, root_ref, cb_ref, foW_ref, fob_ref, o_ref):
        agg = a_ref[0] + a_ref[1]
        deg = jnp.maximum(d_ref[0] + d_ref[1], 1.0)
        rat = agg / deg
        hb = h_ref[...]
        for m in range(8):
            hm = hb[:, H * m:H * (m + 1)]
            am = rat[:, H * m:H * (m + 1)]
            hn = jnp.maximum(hm @ root_ref[...] + cb_ref[...] + am, 0.0)
            o_ref[:, O * m:O * (m + 1)] = hn @ foW_ref[...] + fob_ref[...]

    return pl.pallas_call(
        body,
        out_shape=jax.ShapeDtypeStruct((R, 8 * O), jnp.float32),
    )(h8, aggp8, degp8, root, cb, foW, fob)


def kernel(x, edge_index, edge_attr, conditions, scale, batch,
           ne_W1, ne_b1, ne_W2, ne_b2,
           ce_W1, ce_b1, ce_W2, ce_b2,
           k_W1, k_b1, k_W2, k_b2, k_W3, k_b3,
           root, conv_b, fo_W, fo_b):
    N, _ = x.shape
    E = edge_index.shape[1]
    A = edge_attr.shape[1]
    H = ne_W1.shape[1]
    num_layers = 3

    e_sub, e_group = 125, E // (_NW * 8 * 125)
    npad = 51200
    n_sub, n_group = 100, npad // (_NW * 8 * 100)
    zch, nz = 625, (N // _NS) // 625
    ebr = 1000
    nrp = 6400

    f32 = jnp.float32
    src = edge_index[0]
    dst = edge_index[1]
    src2d = src.reshape(-1, e_sub)
    dst2d = dst.reshape(-1, e_sub)
    batch_p = jnp.pad(batch, (0, npad - N)).reshape(-1, n_sub)

    r1 = lambda b: b.reshape(1, -1)
    cs = jnp.concatenate([conditions, scale], axis=1)

    M = jnp.kron(jnp.eye(H, dtype=f32), jnp.ones((1, H), f32))
    Q = jnp.kron(jnp.ones((H, 1), f32), jnp.eye(H, dtype=f32))

    x8 = jnp.pad(x.reshape(N // 8, -1), ((0, nrp - N // 8), (0, 0)))
    eag = edge_attr.reshape(E // 8, 8 * A)

    u = _tc_cond(cs, ce_W1, r1(ce_b1), ce_W2, r1(ce_b2))
    h8 = _tc_nodes(x8, ne_W1, r1(ne_b1), ne_W2, r1(ne_b2), 400)[:N // 8]

    ubig = _gather_rows(u, batch_p, npad, n_group, n_sub)
    ue = _gather_rows(ubig, src2d, E, e_group, e_sub)
    ue8 = ue.reshape(E // 8, 8 * H)
    degp8 = _degree_partials(dst2d, N, e_group, e_sub, zch, nz, H) \
        .reshape(_NC, N // 8, 8 * H)

    W1a, W1b = k_W1[:A], k_W1[A:]
    for layer in range(num_layers):
        xj = _gather_rows(h8.reshape(N, H), src2d, E, e_group, e_sub)
        msg8 = _tc_msg(eag, ue8, xj.reshape(E // 8, 8 * H), W1a, W1b,
                       r1(k_b1), k_W2, r1(k_b2), k_W3, r1(k_b3), M, Q, ebr)
        aggp8 = _scatter_partials(msg8.reshape(E, H), dst2d, N, e_group,
                                  e_sub, zch, nz).reshape(_NC, N // 8, 8 * H)
        if layer < num_layers - 1:
            h8 = _tc_update(h8, aggp8, degp8, root, r1(conv_b))
        else:
            outp = _tc_update_final(h8, aggp8, degp8, root, r1(conv_b),
                                    fo_W, r1(fo_b))
    out = outp.reshape(N, 1)
    return (out, u)


# speedup vs baseline: 6.8653x; 1.1608x over previous
import functools

import jax
import jax.numpy as jnp
from jax import lax
from jax.experimental import pallas as pl
from jax.experimental.pallas import tpu as pltpu
from jax.experimental.pallas import tpu_sc as plsc

_NC = 2
_NS = 16
_NW = _NC * _NS


def _gather_rows(table, idx, n_rows, group, ksub, sub):
    T, D = table.shape
    G = group
    mesh = plsc.VectorSubcoreMesh(core_axis_name="c", subcore_axis_name="s")

    def body(table_hbm, idx_hbm, out_hbm, idx_st, rows_v, sem):
        c = lax.axis_index("c")
        s = lax.axis_index("s")
        w = s * _NC + c

        def grp(g, carry):
            base = (w * G + g) * ksub * sub
            pltpu.sync_copy(idx_hbm.at[pl.ds(base, ksub * sub)], idx_st)
            descs = [
                pltpu.async_copy(
                    table_hbm.at[idx_st.at[pl.ds(j * sub, sub)]],
                    rows_v.at[pl.ds(j * sub, sub)],
                    sem,
                )
                for j in range(ksub)
            ]
            for d in descs:
                d.wait()
            pltpu.sync_copy(rows_v, out_hbm.at[pl.ds(base, ksub * sub)])
            return carry

        lax.fori_loop(0, G, grp, 0)

    return pl.kernel(
        body,
        out_type=jax.ShapeDtypeStruct((n_rows, D), jnp.float32),
        mesh=mesh,
        scratch_types=[
            pltpu.VMEM((ksub * sub,), jnp.int32),
            pltpu.VMEM((ksub * sub, D), jnp.float32),
            pltpu.SemaphoreType.DMA,
        ],
        compiler_params=pltpu.CompilerParams(use_tc_tiling_on_sc=False),
    )(table, idx)


def _scatter_partials(vals, idx, n_seg, group, ksub, sub, zch, nz):
    E_, D = vals.shape
    G = group
    rpt = n_seg // _NS
    mesh = plsc.VectorSubcoreMesh(core_axis_name="c", subcore_axis_name="s")

    def body(vals_hbm, idx_hbm, out_hbm, idx_st, vals_st, zbuf, acc_sh, sem):
        c = lax.axis_index("c")
        s = lax.axis_index("s")

        def zf(r, carry):
            zbuf[r, :] = jnp.zeros((D,), jnp.float32)
            return carry

        lax.fori_loop(0, zch, zf, 0)
        for k in range(nz):
            pltpu.sync_copy(zbuf, acc_sh.at[pl.ds(s * rpt + k * zch, zch)])
        plsc.subcore_barrier()

        def grp(g, carry):
            base = ((c * _NS + s) * G + g) * ksub * sub
            pltpu.sync_copy(idx_hbm.at[pl.ds(base, ksub * sub)], idx_st)
            pltpu.sync_copy(vals_hbm.at[pl.ds(base, ksub * sub)], vals_st)
            for j in range(ksub):
                pltpu.sync_copy(
                    vals_st.at[pl.ds(j * sub, sub)],
                    acc_sh.at[idx_st.at[pl.ds(j * sub, sub)]],
                    add=True,
                )
            return carry

        lax.fori_loop(0, G, grp, 0)
        plsc.subcore_barrier()
        for k in range(nz):
            pltpu.sync_copy(acc_sh.at[pl.ds(s * rpt + k * zch, zch)], zbuf)
            pltpu.sync_copy(zbuf, out_hbm.at[c, pl.ds(s * rpt + k * zch, zch)])

    return pl.kernel(
        body,
        out_type=jax.ShapeDtypeStruct((_NC, n_seg, D), jnp.float32),
        mesh=mesh,
        scratch_types=[
            pltpu.VMEM((ksub * sub,), jnp.int32),
            pltpu.VMEM((ksub * sub, D), jnp.float32),
            pltpu.VMEM((zch, D), jnp.float32),
            pltpu.VMEM_SHARED((n_seg, D), jnp.float32),
            pltpu.SemaphoreType.DMA,
        ],
        compiler_params=pltpu.CompilerParams(use_tc_tiling_on_sc=False),
    )(vals, idx)


def _degree_partials(idx, n_seg, group, ksub, sub, zch, nz, d):
    G = group
    rpt = n_seg // _NS
    mesh = plsc.VectorSubcoreMesh(core_axis_name="c", subcore_axis_name="s")

    def body(idx_hbm, out_hbm, idx_st, ones_v, zbuf, acc_sh, sem):
        c = lax.axis_index("c")
        s = lax.axis_index("s")

        def of(r, carry):
            ones_v[r, :] = jnp.ones((d,), jnp.float32)
            return carry

        lax.fori_loop(0, sub, of, 0)

        def zf(r, carry):
            zbuf[r, :] = jnp.zeros((d,), jnp.float32)
            return carry

        lax.fori_loop(0, zch, zf, 0)
        for k in range(nz):
            pltpu.sync_copy(zbuf, acc_sh.at[pl.ds(s * rpt + k * zch, zch)])
        plsc.subcore_barrier()

        def grp(g, carry):
            base = ((c * _NS + s) * G + g) * ksub * sub
            pltpu.sync_copy(idx_hbm.at[pl.ds(base, ksub * sub)], idx_st)
            for j in range(ksub):
                pltpu.sync_copy(
                    ones_v, acc_sh.at[idx_st.at[pl.ds(j * sub, sub)]],
                    add=True)
            return carry

        lax.fori_loop(0, G, grp, 0)
        plsc.subcore_barrier()
        for k in range(nz):
            pltpu.sync_copy(acc_sh.at[pl.ds(s * rpt + k * zch, zch)], zbuf)
            pltpu.sync_copy(zbuf, out_hbm.at[c, pl.ds(s * rpt + k * zch, zch)])

    return pl.kernel(
        body,
        out_type=jax.ShapeDtypeStruct((_NC, n_seg, d), jnp.float32),
        mesh=mesh,
        scratch_types=[
            pltpu.VMEM((ksub * sub,), jnp.int32),
            pltpu.VMEM((sub, d), jnp.float32),
            pltpu.VMEM((zch, d), jnp.float32),
            pltpu.VMEM_SHARED((n_seg, d), jnp.float32),
            pltpu.SemaphoreType.DMA,
        ],
        compiler_params=pltpu.CompilerParams(use_tc_tiling_on_sc=False),
    )(idx)


def _tc_cond(cs, W1, b1, W2, b2):
    def body(cs_ref, W1_ref, b1_ref, W2_ref, b2_ref, u_ref):
        t = jnp.maximum(cs_ref[...] @ W1_ref[...] + b1_ref[...], 0.0)
        u_ref[...] = t @ W2_ref[...] + b2_ref[...]

    B, _ = cs.shape
    H = W1.shape[1]
    return pl.pallas_call(
        body, out_shape=jax.ShapeDtypeStruct((B, H), jnp.float32)
    )(cs, W1, b1, W2, b2)


def _tc_nodes(x8, W1, b1, W2, b2, rb):
    R, F8 = x8.shape
    F = F8 // 8
    H = W1.shape[1]

    def body(x_ref, W1_ref, b1_ref, W2_ref, b2_ref, h_ref):
        xb = x_ref[...]
        for m in range(8):
            xm = xb[:, F * m:F * (m + 1)]
            t = jnp.maximum(xm @ W1_ref[...] + b1_ref[...], 0.0)
            h_ref[:, H * m:H * (m + 1)] = t @ W2_ref[...] + b2_ref[...]

    full = lambda i: (0, 0)
    return pl.pallas_call(
        body,
        grid=(R // rb,),
        in_specs=[
            pl.BlockSpec((rb, F8), lambda i: (i, 0)),
            pl.BlockSpec(W1.shape, full),
            pl.BlockSpec(b1.shape, full),
            pl.BlockSpec(W2.shape, full),
            pl.BlockSpec(b2.shape, full),
        ],
        out_specs=pl.BlockSpec((rb, 8 * H), lambda i: (i, 0)),
        out_shape=jax.ShapeDtypeStruct((R, 8 * H), jnp.float32),
    )(x8, W1, b1, W2, b2)


def _tc_pre(eag32, ue32, kW1a32, kW1b32, b1t, pbr):
    R32, _ = eag32.shape

    def body(ea_ref, ue_ref, wa_ref, wb_ref, b_ref, o_ref):
        o_ref[...] = (ea_ref[...] @ wa_ref[...] + ue_ref[...] @ wb_ref[...]
                      + b_ref[...])

    full = lambda i: (0, 0)
    return pl.pallas_call(
        body,
        grid=(R32 // pbr,),
        in_specs=[
            pl.BlockSpec((pbr, 128), lambda i: (i, 0)),
            pl.BlockSpec((pbr, 512), lambda i: (i, 0)),
            pl.BlockSpec(kW1a32.shape, full),
            pl.BlockSpec(kW1b32.shape, full),
            pl.BlockSpec(b1t.shape, full),
        ],
        out_specs=pl.BlockSpec((pbr, 1024), lambda i: (i, 0)),
        out_shape=jax.ShapeDtypeStruct((R32, 1024), jnp.float32),
    )(eag32, ue32, kW1a32, kW1b32, b1t)


def _tc_msg(pre8, xj8, kW2bd, b2t, W3, b3, M, Q, ebr):
    R, _ = pre8.shape
    H = Q.shape[1]
    KH = W3.shape[0]

    def body(pre_ref, xj_ref, W2_ref, b2_ref, W3_ref, b3_ref, M_ref,
             Q_ref, msg_ref):
        kh1 = jnp.maximum(pre_ref[...], 0.0)
        kh2 = jnp.maximum(kh1 @ W2_ref[...] + b2_ref[...], 0.0)
        xjb = xj_ref[...]
        for m in range(8):
            kw = kh2[:, KH * m:KH * (m + 1)] @ W3_ref[...] + b3_ref[...]
            xe = xjb[:, H * m:H * (m + 1)] @ M_ref[...]
            msg_ref[:, H * m:H * (m + 1)] = (xe * kw) @ Q_ref[...]

    full = lambda i: (0, 0)
    return pl.pallas_call(
        body,
        grid=(R // ebr,),
        in_specs=[
            pl.BlockSpec((ebr, 256), lambda i: (i, 0)),
            pl.BlockSpec((ebr, 8 * H), lambda i: (i, 0)),
            pl.BlockSpec(kW2bd.shape, full),
            pl.BlockSpec(b2t.shape, full),
            pl.BlockSpec(W3.shape, full),
            pl.BlockSpec(b3.shape, full),
            pl.BlockSpec(M.shape, full),
            pl.BlockSpec(Q.shape, full),
        ],
        out_specs=pl.BlockSpec((ebr, 8 * H), lambda i: (i, 0)),
        out_shape=jax.ShapeDtypeStruct((R, 8 * H), jnp.float32),
    )(pre8, xj8, kW2bd, b2t, W3, b3, M, Q)


def _tc_update(h8, aggp8, degp8, root, cb):
    R, H8 = h8.shape
    H = H8 // 8

    def body(h_ref, a_ref, d_ref, root_ref, cb_ref, o_ref):
        agg = a_ref[0] + a_ref[1]
        deg = jnp.maximum(d_ref[0] + d_ref[1], 1.0)
        o_ref[...] = jnp.maximum(
            agg / deg + h_ref[...] @ root_ref[...] + cb_ref[...], 0.0)

    return pl.pallas_call(
        body,
        out_shape=jax.ShapeDtypeStruct((R, H8), jnp.float32),
    )(h8, aggp8, degp8, root, cb)


def _tc_update_final(h8, aggp8, degp8, root, cb, foW, fob):
    R, H8 = h8.shape
    O = foW.shape[1]

    def body(h_ref, a_ref, d_ref, root_ref, cb_ref, foW_ref, fob_ref, o_ref):
        agg = a_ref[0] + a_ref[1]
        deg = jnp.maximum(d_ref[0] + d_ref[1], 1.0)
        hn = jnp.maximum(
            agg / deg + h_ref[...] @ root_ref[...] + cb_ref[...], 0.0)
        o_ref[...] = hn @ foW_ref[...] + fob_ref[...]

    return pl.pallas_call(
        body,
        out_shape=jax.ShapeDtypeStruct((R, O), jnp.float32),
    )(h8, aggp8, degp8, root, cb, foW, fob)


def kernel(x, edge_index, edge_attr, conditions, scale, batch,
           ne_W1, ne_b1, ne_W2, ne_b2,
           ce_W1, ce_b1, ce_W2, ce_b2,
           k_W1, k_b1, k_W2, k_b2, k_W3, k_b3,
           root, conv_b, fo_W, fo_b):
    N, _ = x.shape
    E = edge_index.shape[1]
    A = edge_attr.shape[1]
    H = ne_W1.shape[1]
    num_layers = 3

    e_sub, e_k = 40, 25
    e_group = E // (_NW * e_k * e_sub)
    npad = 51200
    n_sub, n_k = 40, 20
    n_group = npad // (_NW * n_k * n_sub)
    zch, nz = 625, (N // _NS) // 625
    ebr = 1000
    nrp = 6400

    f32 = jnp.float32
    src = edge_index[0]
    dst = edge_index[1]
    batch_p = jnp.pad(batch, (0, npad - N))

    r1 = lambda b: b.reshape(1, -1)
    cs = jnp.concatenate([conditions, scale], axis=1)

    M = jnp.kron(jnp.eye(H, dtype=f32), jnp.ones((1, H), f32))
    Q = jnp.kron(jnp.ones((H, 1), f32), jnp.eye(H, dtype=f32))

    W1a, W1b = k_W1[:A], k_W1[A:]
    kW1a32 = jnp.kron(jnp.eye(32, dtype=f32), W1a)
    kW1b32 = jnp.kron(jnp.eye(32, dtype=f32), W1b)
    b1t32 = jnp.tile(k_b1, 32).reshape(1, -1)
    kW2bd = jnp.kron(jnp.eye(8, dtype=f32), k_W2)
    b2t = jnp.tile(k_b2, 8).reshape(1, -1)
    kroot = jnp.kron(jnp.eye(8, dtype=f32), root)
    cbt = jnp.tile(conv_b, 8).reshape(1, -1)
    kfoW = jnp.kron(jnp.eye(8, dtype=f32), fo_W)
    fobt = jnp.tile(fo_b, 8).reshape(1, -1)

    x8 = jnp.pad(x.reshape(N // 8, -1), ((0, nrp - N // 8), (0, 0)))
    eag32 = edge_attr.reshape(E // 32, 128)

    u = _tc_cond(cs, ce_W1, r1(ce_b1), ce_W2, r1(ce_b2))
    h8 = _tc_nodes(x8, ne_W1, r1(ne_b1), ne_W2, r1(ne_b2), 400)[:N // 8]

    ubig = _gather_rows(u, batch_p, npad, n_group, n_k, n_sub)
    ue = _gather_rows(ubig, src, E, e_group, e_k, e_sub)
    pre8 = _tc_pre(eag32, ue.reshape(E // 32, 512), kW1a32, kW1b32,
                   b1t32, 1000).reshape(E // 8, 256)
    degp8 = _degree_partials(dst, N, e_group, e_k, e_sub, zch, nz, H) \
        .reshape(_NC, N // 8, 8 * H)

    for layer in range(num_layers):
        xj = _gather_rows(h8.reshape(N, H), src, E, e_group, e_k, e_sub)
        msg8 = _tc_msg(pre8, xj.reshape(E // 8, 8 * H), kW2bd, b2t,
                       k_W3, r1(k_b3), M, Q, ebr)
        aggp8 = _scatter_partials(msg8.reshape(E, H), dst, N, e_group,
                                  e_k, e_sub, zch, nz) \
            .reshape(_NC, N // 8, 8 * H)
        if layer < num_layers - 1:
            h8 = _tc_update(h8, aggp8, degp8, kroot, cbt)
        else:
            outp = _tc_update_final(h8, aggp8, degp8, kroot, cbt,
                                    kfoW, fobt)
    out = outp.reshape(N, 1)
    return (out, u)
